# baseline (speedup 1.0000x reference)
import functools

import jax
import jax.numpy as jnp
from jax import lax
from jax.experimental import pallas as pl
from jax.experimental.pallas import tpu as pltpu
from jax.experimental.pallas import tpu_sc as plsc

_EPS = 1e-5
_NS = 16
_K = 128


def _sc_degree(NP, C2, Er):
    slab = NP // _NS
    mesh = plsc.VectorSubcoreMesh(core_axis_name="c", subcore_axis_name="s")

    @functools.partial(
        pl.kernel,
        out_type=jax.ShapeDtypeStruct((2 * NP, 16), jnp.float32),
        mesh=mesh,
        scratch_types=[
            pltpu.VMEM((C2, _K), jnp.int32),
            pltpu.VMEM((_K, 16), jnp.float32),
            pltpu.VMEM_SHARED((NP, 16), jnp.float32),
        ],
    )
    def deg_kernel(dstb, out, dst_v, stage_v, hist_sp):
        c = lax.axis_index("c")
        s = lax.axis_index("s")
        wid = c * _NS + s

        def zrow(i, carry):
            stage_v[i, :] = jnp.zeros((16,), jnp.float32)
            return carry

        lax.fori_loop(0, _K, zrow, 0)
        for k in range(slab // _K):
            pltpu.sync_copy(stage_v, hist_sp.at[pl.ds(s * slab + k * _K, _K)])

        def orow(i, carry):
            stage_v[i, :] = jnp.ones((16,), jnp.float32)
            return carry

        lax.fori_loop(0, _K, orow, 0)
        pltpu.sync_copy(dstb.at[pl.ds(wid * C2, C2)], dst_v)
        plsc.subcore_barrier()

        def body(g, carry):
            pltpu.sync_copy(stage_v, hist_sp.at[dst_v.at[g]], add=True)
            return carry

        lax.fori_loop(0, C2, body, 0)
        plsc.subcore_barrier()
        pltpu.sync_copy(
            hist_sp.at[pl.ds(s * slab, slab)],
            out.at[pl.ds(c * NP + s * slab, slab)],
        )

    return deg_kernel


def _sc_edge_agg(NP, C, Er):
    slab = NP // _NS
    B = C // 16
    mesh = plsc.VectorSubcoreMesh(core_axis_name="c", subcore_axis_name="s")

    @functools.partial(
        pl.kernel,
        out_type=jax.ShapeDtypeStruct((2 * NP, _K), jnp.float32),
        mesh=mesh,
        scratch_types=[
            pltpu.VMEM((16, _K), jnp.int32),
            pltpu.VMEM((16, _K), jnp.int32),
            pltpu.VMEM((_K, _K), jnp.float32),
            pltpu.VMEM((_K, _K), jnp.float32),
            pltpu.VMEM_SHARED((NP, _K), jnp.float32),
            pltpu.SemaphoreType.DMA,
            pltpu.SemaphoreType.DMA,
            pltpu.SemaphoreType.DMA,
            pltpu.SemaphoreType.DMA,
        ],
    )
    def agg_kernel(table, srcb2, dstb, out, sidx, didx, rows0, rows1,
                   acc_sp, gsem0, gsem1, gsem2, gsem3):
        c = lax.axis_index("c")
        s = lax.axis_index("s")
        rows = (rows0, rows1)
        gsemA = (gsem0, gsem1)
        gsemB = (gsem2, gsem3)
        _H = _K // 2

        def start_gather(r, p):
            pltpu.async_copy(
                table.at[sidx.at[r, pl.ds(0, _H)]],
                rows[p].at[pl.ds(0, _H)],
                gsemA[p],
            )
            pltpu.async_copy(
                table.at[sidx.at[r, pl.ds(_H, _H)]],
                rows[p].at[pl.ds(_H, _H)],
                gsemB[p],
            )

        def wait_gather(p):
            pltpu.make_async_copy(
                table.at[pl.ds(0, _H)], rows[p].at[pl.ds(0, _H)], gsemA[p]
            ).wait()
            pltpu.make_async_copy(
                table.at[pl.ds(0, _H)], rows[p].at[pl.ds(_H, _H)], gsemB[p]
            ).wait()

        def scatter(r, p):
            pltpu.sync_copy(rows[p], acc_sp.at[didx.at[r]], add=True)

        def zrow(i, carry):
            for j in range(8):
                rows0[i, pl.ds(j * 16, 16)] = jnp.zeros((16,), jnp.float32)
            return carry

        lax.fori_loop(0, _K, zrow, 0)
        for k in range(slab // _K):
            pltpu.sync_copy(rows0, acc_sp.at[pl.ds(s * slab + k * _K, _K)])
        plsc.subcore_barrier()

        def load_idx(base):
            pltpu.sync_copy(srcb2.at[pl.ds(c * Er + base, 16)], sidx)
            pltpu.sync_copy(dstb.at[pl.ds(base, 16)], didx)

        def do_block():
            start_gather(0, 0)
            for r in range(16):
                p = r & 1
                wait_gather(p)
                if r + 1 < 16:
                    start_gather(r + 1, 1 - p)
                scatter(r, p)

        load_idx(s * C)
        do_block()

        def blk(b, carry):
            load_idx(s * C + b * 16)
            do_block()
            return carry

        lax.fori_loop(1, B, blk, 0)
        plsc.subcore_barrier()
        pltpu.sync_copy(
            acc_sp.at[pl.ds(s * slab, slab)],
            out.at[pl.ds(c * NP + s * slab, slab)],
        )

    return agg_kernel


def _tc_proj(NP, BN, N, D_IN, DH):
    NB = NP // BN

    def body(x_ref, w_ref, b_ref, d0_ref, d1_ref, h_ref, dinv_ref):
        nb = pl.program_id(0)
        h = jnp.dot(x_ref[...], w_ref[...], preferred_element_type=jnp.float32)
        h = jnp.maximum(h + b_ref[...], 0.0)
        rows = nb * BN + lax.broadcasted_iota(jnp.int32, (BN, 1), 0)
        mask = (rows < N).astype(jnp.float32)
        h_ref[...] = h * mask
        d = d0_ref[:, 0:1] + d1_ref[:, 0:1] + 1.0
        dinv_ref[...] = lax.rsqrt(d)

    return pl.pallas_call(
        body,
        grid=(NB,),
        in_specs=[
            pl.BlockSpec((BN, D_IN), lambda nb: (nb, 0)),
            pl.BlockSpec((D_IN, DH), lambda nb: (0, 0)),
            pl.BlockSpec((1, DH), lambda nb: (0, 0)),
            pl.BlockSpec((BN, 16), lambda nb: (nb, 0)),
            pl.BlockSpec((BN, 16), lambda nb: (NB + nb, 0)),
        ],
        out_specs=[
            pl.BlockSpec((BN, DH), lambda nb: (nb, 0)),
            pl.BlockSpec((BN, 1), lambda nb: (nb, 0)),
        ],
        out_shape=[
            jax.ShapeDtypeStruct((NP, DH), jnp.float32),
            jax.ShapeDtypeStruct((NP, 1), jnp.float32),
        ],
    )


def _tc_mm_table(NP, BN, N, DH):
    NB = NP // BN
    HD = DH // 2

    def body(h_ref, w_ref, dinv_ref, t_ref):
        nb = pl.program_id(1)
        m = jnp.dot(h_ref[...], w_ref[...], preferred_element_type=jnp.float32)
        rows = nb * BN + lax.broadcasted_iota(jnp.int32, (BN, 1), 0)
        mask = (rows < N).astype(jnp.float32)
        t_ref[...] = m * dinv_ref[...] * mask

    return pl.pallas_call(
        body,
        grid=(2, NB),
        in_specs=[
            pl.BlockSpec((BN, DH), lambda h, nb: (nb, 0)),
            pl.BlockSpec((DH, HD), lambda h, nb: (0, h)),
            pl.BlockSpec((BN, 1), lambda h, nb: (nb, 0)),
        ],
        out_specs=pl.BlockSpec((BN, HD), lambda h, nb: (h * NB + nb, 0)),
        out_shape=jax.ShapeDtypeStruct((2 * NP, HD), jnp.float32),
    )


def _tc_stats(NP, BN, N, DH):
    NB = NP // BN
    HD = DH // 2

    def body(S_ref, t_ref, dinv_ref, bc_ref, agg_ref, s1_ref, s2_ref):
        nb = pl.program_id(1)
        a = (S_ref[...] + t_ref[...]) * dinv_ref[...] + bc_ref[...]
        rows = nb * BN + lax.broadcasted_iota(jnp.int32, (BN, 1), 0)
        mask = (rows < N).astype(jnp.float32)
        a = a * mask
        agg_ref[...] = a

        @pl.when(nb == 0)
        def _():
            s1_ref[...] = jnp.zeros_like(s1_ref)
            s2_ref[...] = jnp.zeros_like(s2_ref)

        s1_ref[0:1, :] = s1_ref[0:1, :] + jnp.sum(a, axis=0, keepdims=True)
        s2_ref[0:1, :] = s2_ref[0:1, :] + jnp.sum(a * a, axis=0, keepdims=True)

    return pl.pallas_call(
        body,
        grid=(2, NB),
        in_specs=[
            pl.BlockSpec((BN, HD), lambda h, nb: (h * NB + nb, 0)),
            pl.BlockSpec((BN, HD), lambda h, nb: (h * NB + nb, 0)),
            pl.BlockSpec((BN, 1), lambda h, nb: (nb, 0)),
            pl.BlockSpec((1, HD), lambda h, nb: (0, h)),
        ],
        out_specs=[
            pl.BlockSpec((BN, HD), lambda h, nb: (nb, h)),
            pl.BlockSpec((8, HD), lambda h, nb: (0, h)),
            pl.BlockSpec((8, HD), lambda h, nb: (0, h)),
        ],
        out_shape=[
            jax.ShapeDtypeStruct((NP, DH), jnp.float32),
            jax.ShapeDtypeStruct((8, DH), jnp.float32),
            jax.ShapeDtypeStruct((8, DH), jnp.float32),
        ],
    )


def _tc_norm(NP, BN, N, DH, residual):
    NB = NP // BN

    def body(agg_ref, s1_ref, s2_ref, g_ref, b_ref, *rest):
        if residual:
            hp_ref, out_ref = rest
        else:
            (out_ref,) = rest
        nb = pl.program_id(0)
        inv_n = 1.0 / N
        mu = jnp.sum(s1_ref[...], axis=0, keepdims=True) * inv_n
        m2 = jnp.sum(s2_ref[...], axis=0, keepdims=True) * inv_n
        var = m2 - mu * mu
        isd = lax.rsqrt(var + _EPS)
        y = (agg_ref[...] - mu) * isd * g_ref[...] + b_ref[...]
        y = jnp.maximum(y, 0.0)
        if residual:
            y = y + hp_ref[...]
        rows = nb * BN + lax.broadcasted_iota(jnp.int32, (BN, 1), 0)
        mask = (rows < N).astype(jnp.float32)
        out_ref[...] = y * mask

    in_specs = [
        pl.BlockSpec((BN, DH), lambda nb: (nb, 0)),
        pl.BlockSpec((8, DH), lambda nb: (0, 0)),
        pl.BlockSpec((8, DH), lambda nb: (0, 0)),
        pl.BlockSpec((1, DH), lambda nb: (0, 0)),
        pl.BlockSpec((1, DH), lambda nb: (0, 0)),
    ]
    if residual:
        in_specs.append(pl.BlockSpec((BN, DH), lambda nb: (nb, 0)))
    return pl.pallas_call(
        body,
        grid=(NB,),
        in_specs=in_specs,
        out_specs=pl.BlockSpec((BN, DH), lambda nb: (nb, 0)),
        out_shape=jax.ShapeDtypeStruct((NP, DH), jnp.float32),
    )


def kernel(x, edge_index, W0, b0, Wc, bc, gamma, beta):
    N, D_IN = x.shape
    DH = W0.shape[1]
    L = Wc.shape[0]
    E = edge_index.shape[1]

    BN = 512
    NP = -(-(N + 1) // 2048) * 2048

    C = 16 * (-(-E // (16 * _NS * _K)))
    Er = _NS * C
    Ep = Er * _K
    C2 = C // 2

    src = edge_index[0]
    dst = edge_index[1]
    pad = jnp.full((Ep - E,), N, dtype=jnp.int32)
    srcb = jnp.concatenate([src, pad]).reshape(Er, _K)
    dstb = jnp.concatenate([dst, pad]).reshape(Er, _K)
    srcb2 = jnp.concatenate([srcb, srcb + NP])
    x_pad = jnp.pad(x, ((0, NP - N), (0, 0)))

    deg = _sc_degree(NP, C2, Er)(dstb)
    h, dinv = _tc_proj(NP, BN, N, D_IN, DH)(
        x_pad, W0, b0.reshape(1, DH), deg, deg
    )

    mm = _tc_mm_table(NP, BN, N, DH)
    agg_sc = _sc_edge_agg(NP, C, Er)
    stats = _tc_stats(NP, BN, N, DH)
    norm0 = _tc_norm(NP, BN, N, DH, residual=False)
    normr = _tc_norm(NP, BN, N, DH, residual=True)

    for i in range(L):
        table = mm(h, Wc[i], dinv)
        S = agg_sc(table, srcb2, dstb)
        agg, s1, s2 = stats(S, table, dinv, bc[i].reshape(1, DH))
        g_i = gamma[i].reshape(1, DH)
        b_i = beta[i].reshape(1, DH)
        if i == 0:
            h = norm0(agg, s1, s2, g_i, b_i)
        else:
            h = normr(agg, s1, s2, g_i, b_i, h)
    return h[:N]


# speedup vs baseline: 1.0151x; 1.0151x over previous
import functools

import jax
import jax.numpy as jnp
from jax import lax
from jax.experimental import pallas as pl
from jax.experimental.pallas import tpu as pltpu
from jax.experimental.pallas import tpu_sc as plsc

_EPS = 1e-5
_NS = 16
_K = 128


def _sc_degree(NP, C2, Er):
    slab = NP // _NS
    mesh = plsc.VectorSubcoreMesh(core_axis_name="c", subcore_axis_name="s")

    @functools.partial(
        pl.kernel,
        out_type=jax.ShapeDtypeStruct((2 * NP, 16), jnp.float32),
        mesh=mesh,
        scratch_types=[
            pltpu.VMEM((C2, _K), jnp.int32),
            pltpu.VMEM((_K, 16), jnp.float32),
            pltpu.VMEM_SHARED((NP, 16), jnp.float32),
        ],
    )
    def deg_kernel(dstb, out, dst_v, stage_v, hist_sp):
        c = lax.axis_index("c")
        s = lax.axis_index("s")
        wid = c * _NS + s

        def zrow(i, carry):
            stage_v[i, :] = jnp.zeros((16,), jnp.float32)
            return carry

        lax.fori_loop(0, _K, zrow, 0)
        for k in range(slab // _K):
            pltpu.sync_copy(stage_v, hist_sp.at[pl.ds(s * slab + k * _K, _K)])

        def orow(i, carry):
            stage_v[i, :] = jnp.ones((16,), jnp.float32)
            return carry

        lax.fori_loop(0, _K, orow, 0)
        pltpu.sync_copy(dstb.at[pl.ds(wid * C2, C2)], dst_v)
        plsc.subcore_barrier()

        def body(g, carry):
            pltpu.sync_copy(stage_v, hist_sp.at[dst_v.at[g]], add=True)
            return carry

        lax.fori_loop(0, C2, body, 0)
        plsc.subcore_barrier()
        pltpu.sync_copy(
            hist_sp.at[pl.ds(s * slab, slab)],
            out.at[pl.ds(c * NP + s * slab, slab)],
        )

    return deg_kernel


def _sc_edge_agg(NP, C, Er):
    slab = NP // _NS
    B = C // 16
    mesh = plsc.VectorSubcoreMesh(core_axis_name="c", subcore_axis_name="s")

    @functools.partial(
        pl.kernel,
        out_type=jax.ShapeDtypeStruct((2 * NP, _K), jnp.float32),
        mesh=mesh,
        scratch_types=[
            pltpu.VMEM((16, _K), jnp.int32),
            pltpu.VMEM((16, _K), jnp.int32),
            pltpu.VMEM((16, _K), jnp.int32),
            pltpu.VMEM((16, _K), jnp.int32),
            pltpu.VMEM((_K, _K), jnp.float32),
            pltpu.VMEM((_K, _K), jnp.float32),
            pltpu.VMEM_SHARED((NP, _K), jnp.float32),
            pltpu.SemaphoreType.DMA,
            pltpu.SemaphoreType.DMA,
            pltpu.SemaphoreType.DMA,
            pltpu.SemaphoreType.DMA,
            pltpu.SemaphoreType.DMA,
            pltpu.SemaphoreType.DMA,
        ],
    )
    def agg_kernel(table, srcb2, dstb, out, sidx0, sidx1, didx0, didx1,
                   rows0, rows1, acc_sp, gsem0, gsem1,
                   isem_s0, isem_s1, isem_d0, isem_d1):
        c = lax.axis_index("c")
        s = lax.axis_index("s")
        rows = (rows0, rows1)
        gsem = (gsem0, gsem1)
        sidxs = (sidx0, sidx1)
        didxs = (didx0, didx1)
        isem_s = (isem_s0, isem_s1)
        isem_d = (isem_d0, isem_d1)

        def start_gather(q, r, p):
            pltpu.async_copy(table.at[sidxs[q].at[r]], rows[p], gsem[p])

        def wait_gather(p):
            pltpu.make_async_copy(table.at[pl.ds(0, _K)], rows[p], gsem[p]).wait()

        def scatter(q, r, p):
            pltpu.sync_copy(rows[p], acc_sp.at[didxs[q].at[r]], add=True)

        def zrow(i, carry):
            for j in range(8):
                rows0[i, pl.ds(j * 16, 16)] = jnp.zeros((16,), jnp.float32)
            return carry

        lax.fori_loop(0, _K, zrow, 0)
        for k in range(slab // _K):
            pltpu.sync_copy(rows0, acc_sp.at[pl.ds(s * slab + k * _K, _K)])
        plsc.subcore_barrier()

        def load_idx_async(base, q):
            pltpu.async_copy(
                srcb2.at[pl.ds(c * Er + base, 16)], sidxs[q], isem_s[q]
            )
            pltpu.async_copy(dstb.at[pl.ds(base, 16)], didxs[q], isem_d[q])

        def wait_idx(q):
            pltpu.make_async_copy(
                srcb2.at[pl.ds(0, 16)], sidxs[q], isem_s[q]
            ).wait()
            pltpu.make_async_copy(
                dstb.at[pl.ds(0, 16)], didxs[q], isem_d[q]
            ).wait()

        def do_block(q):
            start_gather(q, 0, 0)
            for r in range(16):
                p = r & 1
                wait_gather(p)
                if r + 1 < 16:
                    start_gather(q, r + 1, 1 - p)
                scatter(q, r, p)

        base0 = s * C
        load_idx_async(base0, 0)
        if B > 1:
            load_idx_async(base0 + 16, 1)

        def pair(i, carry):
            b = 2 * i
            wait_idx(0)
            do_block(0)
            load_idx_async(base0 + lax.rem(b + 2, B) * 16, 0)
            wait_idx(1)
            do_block(1)
            load_idx_async(base0 + lax.rem(b + 3, B) * 16, 1)
            return carry

        if B > 1:
            lax.fori_loop(0, B // 2, pair, 0)
        if B % 2 == 1:
            wait_idx(0)
            do_block(0)
            if B > 1:
                wait_idx(1)
        else:
            wait_idx(0)
            wait_idx(1)
        plsc.subcore_barrier()
        pltpu.sync_copy(
            acc_sp.at[pl.ds(s * slab, slab)],
            out.at[pl.ds(c * NP + s * slab, slab)],
        )

    return agg_kernel


def _tc_proj(NP, BN, N, D_IN, DH):
    NB = NP // BN

    def body(x_ref, w_ref, b_ref, d0_ref, d1_ref, h_ref, dinv_ref):
        nb = pl.program_id(0)
        h = jnp.dot(x_ref[...], w_ref[...], preferred_element_type=jnp.float32)
        h = jnp.maximum(h + b_ref[...], 0.0)
        rows = nb * BN + lax.broadcasted_iota(jnp.int32, (BN, 1), 0)
        mask = (rows < N).astype(jnp.float32)
        h_ref[...] = h * mask
        d = d0_ref[:, 0:1] + d1_ref[:, 0:1] + 1.0
        dinv_ref[...] = lax.rsqrt(d)

    return pl.pallas_call(
        body,
        grid=(NB,),
        in_specs=[
            pl.BlockSpec((BN, D_IN), lambda nb: (nb, 0)),
            pl.BlockSpec((D_IN, DH), lambda nb: (0, 0)),
            pl.BlockSpec((1, DH), lambda nb: (0, 0)),
            pl.BlockSpec((BN, 16), lambda nb: (nb, 0)),
            pl.BlockSpec((BN, 16), lambda nb: (NB + nb, 0)),
        ],
        out_specs=[
            pl.BlockSpec((BN, DH), lambda nb: (nb, 0)),
            pl.BlockSpec((BN, 1), lambda nb: (nb, 0)),
        ],
        out_shape=[
            jax.ShapeDtypeStruct((NP, DH), jnp.float32),
            jax.ShapeDtypeStruct((NP, 1), jnp.float32),
        ],
    )


def _tc_mm_table(NP, BN, N, DH):
    NB = NP // BN
    HD = DH // 2

    def body(h_ref, w_ref, dinv_ref, t_ref):
        nb = pl.program_id(1)
        m = jnp.dot(h_ref[...], w_ref[...], preferred_element_type=jnp.float32)
        rows = nb * BN + lax.broadcasted_iota(jnp.int32, (BN, 1), 0)
        mask = (rows < N).astype(jnp.float32)
        t_ref[...] = m * dinv_ref[...] * mask

    return pl.pallas_call(
        body,
        grid=(2, NB),
        in_specs=[
            pl.BlockSpec((BN, DH), lambda h, nb: (nb, 0)),
            pl.BlockSpec((DH, HD), lambda h, nb: (0, h)),
            pl.BlockSpec((BN, 1), lambda h, nb: (nb, 0)),
        ],
        out_specs=pl.BlockSpec((BN, HD), lambda h, nb: (h * NB + nb, 0)),
        out_shape=jax.ShapeDtypeStruct((2 * NP, HD), jnp.float32),
    )


def _tc_stats(NP, BN, N, DH):
    NB = NP // BN
    HD = DH // 2

    def body(S_ref, t_ref, dinv_ref, bc_ref, agg_ref, s1_ref, s2_ref):
        nb = pl.program_id(1)
        a = (S_ref[...] + t_ref[...]) * dinv_ref[...] + bc_ref[...]
        rows = nb * BN + lax.broadcasted_iota(jnp.int32, (BN, 1), 0)
        mask = (rows < N).astype(jnp.float32)
        a = a * mask
        agg_ref[...] = a

        @pl.when(nb == 0)
        def _():
            s1_ref[...] = jnp.zeros_like(s1_ref)
            s2_ref[...] = jnp.zeros_like(s2_ref)

        s1_ref[0:1, :] = s1_ref[0:1, :] + jnp.sum(a, axis=0, keepdims=True)
        s2_ref[0:1, :] = s2_ref[0:1, :] + jnp.sum(a * a, axis=0, keepdims=True)

    return pl.pallas_call(
        body,
        grid=(2, NB),
        in_specs=[
            pl.BlockSpec((BN, HD), lambda h, nb: (h * NB + nb, 0)),
            pl.BlockSpec((BN, HD), lambda h, nb: (h * NB + nb, 0)),
            pl.BlockSpec((BN, 1), lambda h, nb: (nb, 0)),
            pl.BlockSpec((1, HD), lambda h, nb: (0, h)),
        ],
        out_specs=[
            pl.BlockSpec((BN, HD), lambda h, nb: (nb, h)),
            pl.BlockSpec((8, HD), lambda h, nb: (0, h)),
            pl.BlockSpec((8, HD), lambda h, nb: (0, h)),
        ],
        out_shape=[
            jax.ShapeDtypeStruct((NP, DH), jnp.float32),
            jax.ShapeDtypeStruct((8, DH), jnp.float32),
            jax.ShapeDtypeStruct((8, DH), jnp.float32),
        ],
    )


def _tc_norm(NP, BN, N, DH, residual):
    NB = NP // BN

    def body(agg_ref, s1_ref, s2_ref, g_ref, b_ref, *rest):
        if residual:
            hp_ref, out_ref = rest
        else:
            (out_ref,) = rest
        nb = pl.program_id(0)
        inv_n = 1.0 / N
        mu = jnp.sum(s1_ref[...], axis=0, keepdims=True) * inv_n
        m2 = jnp.sum(s2_ref[...], axis=0, keepdims=True) * inv_n
        var = m2 - mu * mu
        isd = lax.rsqrt(var + _EPS)
        y = (agg_ref[...] - mu) * isd * g_ref[...] + b_ref[...]
        y = jnp.maximum(y, 0.0)
        if residual:
            y = y + hp_ref[...]
        rows = nb * BN + lax.broadcasted_iota(jnp.int32, (BN, 1), 0)
        mask = (rows < N).astype(jnp.float32)
        out_ref[...] = y * mask

    in_specs = [
        pl.BlockSpec((BN, DH), lambda nb: (nb, 0)),
        pl.BlockSpec((8, DH), lambda nb: (0, 0)),
        pl.BlockSpec((8, DH), lambda nb: (0, 0)),
        pl.BlockSpec((1, DH), lambda nb: (0, 0)),
        pl.BlockSpec((1, DH), lambda nb: (0, 0)),
    ]
    if residual:
        in_specs.append(pl.BlockSpec((BN, DH), lambda nb: (nb, 0)))
    return pl.pallas_call(
        body,
        grid=(NB,),
        in_specs=in_specs,
        out_specs=pl.BlockSpec((BN, DH), lambda nb: (nb, 0)),
        out_shape=jax.ShapeDtypeStruct((NP, DH), jnp.float32),
    )


def kernel(x, edge_index, W0, b0, Wc, bc, gamma, beta):
    N, D_IN = x.shape
    DH = W0.shape[1]
    L = Wc.shape[0]
    E = edge_index.shape[1]

    BN = 512
    NP = -(-(N + 1) // 2048) * 2048

    C = 16 * (-(-E // (16 * _NS * _K)))
    Er = _NS * C
    Ep = Er * _K
    C2 = C // 2

    src = edge_index[0]
    dst = edge_index[1]
    pad = jnp.full((Ep - E,), N, dtype=jnp.int32)
    srcb = jnp.concatenate([src, pad]).reshape(Er, _K)
    dstb = jnp.concatenate([dst, pad]).reshape(Er, _K)
    srcb2 = jnp.concatenate([srcb, srcb + NP])
    x_pad = jnp.pad(x, ((0, NP - N), (0, 0)))

    deg = _sc_degree(NP, C2, Er)(dstb)
    h, dinv = _tc_proj(NP, BN, N, D_IN, DH)(
        x_pad, W0, b0.reshape(1, DH), deg, deg
    )

    mm = _tc_mm_table(NP, BN, N, DH)
    agg_sc = _sc_edge_agg(NP, C, Er)
    stats = _tc_stats(NP, BN, N, DH)
    norm0 = _tc_norm(NP, BN, N, DH, residual=False)
    normr = _tc_norm(NP, BN, N, DH, residual=True)

    for i in range(L):
        table = mm(h, Wc[i], dinv)
        S = agg_sc(table, srcb2, dstb)
        agg, s1, s2 = stats(S, table, dinv, bc[i].reshape(1, DH))
        g_i = gamma[i].reshape(1, DH)
        b_i = beta[i].reshape(1, DH)
        if i == 0:
            h = norm0(agg, s1, s2, g_i, b_i)
        else:
            h = normr(agg, s1, s2, g_i, b_i, h)
    return h[:N]
